# baseline (speedup 1.0000x reference)
import functools

import jax
import jax.numpy as jnp
from jax import lax
from jax.experimental import pallas as pl
from jax.experimental.pallas import tpu as pltpu
from jax.experimental.pallas import tpu_sc as plsc

HASH_N = 20480
EMB = 128
DM = 2048
P1C = 31337

BATCH = 4
SEQ = 4096
NTOK = BATCH * SEQ
NWORK = 32
CHUNK = NTOK // NWORK
GROUPS = CHUNK // 16
ROWS_PER_DMA = 128
NDMA = CHUNK // ROWS_PER_DMA

BT = 512


def _sc_gather_kernel(ids_hbm, table_hbm, emb_hbm, ids_v, h_v, rows_v, sem):
    wid = lax.axis_index("s") * 2 + lax.axis_index("c")
    base = wid * CHUNK

    @pl.when(wid % (SEQ // CHUNK) == 0)
    def _():
        ids_v[pl.ds(0, 16)] = jnp.zeros((16,), jnp.int32)
        pltpu.sync_copy(ids_hbm.at[pl.ds(base, CHUNK)], ids_v.at[pl.ds(8, CHUNK)])

    @pl.when(wid % (SEQ // CHUNK) != 0)
    def _():
        pltpu.sync_copy(ids_hbm.at[pl.ds(base - 8, CHUNK + 8)], ids_v)

    for g in range(GROUPS):
        cur = ids_v[pl.ds(8 + g * 16, 16)]
        prev = ids_v[pl.ds(7 + g * 16, 16)]
        x = (prev * P1C) ^ cur
        r = lax.rem(x, HASH_N)
        h = jnp.where(r < 0, r + HASH_N, r)
        h_v[g // (ROWS_PER_DMA // 16), pl.ds((g % (ROWS_PER_DMA // 16)) * 16, 16)] = h

    cps = [
        pltpu.async_copy(
            table_hbm.at[h_v.at[j]],
            rows_v.at[pl.ds(j * ROWS_PER_DMA, ROWS_PER_DMA)],
            sem,
        )
        for j in range(NDMA)
    ]
    for cp in cps:
        cp.wait()
    pltpu.sync_copy(rows_v, emb_hbm.at[pl.ds(base, CHUNK)])


def _sc_gather(ids_flat, table):
    mesh = plsc.VectorSubcoreMesh(core_axis_name="c", subcore_axis_name="s")
    fn = functools.partial(
        pl.kernel,
        mesh=mesh,
        out_type=jax.ShapeDtypeStruct((NTOK, EMB), jnp.float32),
        scratch_types=[
            pltpu.VMEM((CHUNK + 8,), jnp.int32),
            pltpu.VMEM((NDMA, ROWS_PER_DMA), jnp.int32),
            pltpu.VMEM((CHUNK, EMB), jnp.float32),
            pltpu.SemaphoreType.DMA,
        ],
    )(_sc_gather_kernel)
    return fn(ids_flat, table)


def _mm_body(x_ref, w_ref, o_ref):
    o_ref[...] = lax.dot_general(
        x_ref[...].astype(jnp.bfloat16),
        w_ref[...].astype(jnp.bfloat16),
        dimension_numbers=(((1,), (1,)), ((), ())),
        preferred_element_type=jnp.float32,
    )


def _project(emb, proj_w):
    return pl.pallas_call(
        _mm_body,
        grid=(NTOK // BT,),
        in_specs=[
            pl.BlockSpec((BT, EMB), lambda i: (i, 0)),
            pl.BlockSpec((DM, EMB), lambda i: (0, 0)),
        ],
        out_specs=pl.BlockSpec((BT, DM), lambda i: (i, 0)),
        out_shape=jax.ShapeDtypeStruct((NTOK, DM), jnp.float32),
    )(emb, proj_w)


@jax.jit
def kernel(input_ids, bigram_emb, proj_w):
    ids_flat = input_ids.reshape(-1)
    emb = _sc_gather(ids_flat, bigram_emb)
    out = _project(emb, proj_w)
    return out.reshape(BATCH, SEQ, DM)


# speedup vs baseline: 1.0806x; 1.0806x over previous
import functools

import jax
import jax.numpy as jnp
from jax import lax
from jax.experimental import pallas as pl
from jax.experimental.pallas import tpu as pltpu
from jax.experimental.pallas import tpu_sc as plsc

HASH_N = 20480
EMB = 128
DM = 2048
P1C = 31337

BATCH = 4
SEQ = 4096
NTOK = BATCH * SEQ
NWORK = 32
CHUNK = NTOK // NWORK
GROUPS = CHUNK // 16
ROWS_PER_DMA = 128
NDMA = CHUNK // ROWS_PER_DMA

BT = 2048


def _sc_gather_kernel(ids_hbm, table_hbm, emb_hbm, ids_v, h_v, rows_v, sem):
    wid = lax.axis_index("s") * 2 + lax.axis_index("c")
    base = wid * CHUNK

    @pl.when(wid % (SEQ // CHUNK) == 0)
    def _():
        ids_v[pl.ds(0, 16)] = jnp.zeros((16,), jnp.int32)
        pltpu.sync_copy(ids_hbm.at[pl.ds(base, CHUNK)], ids_v.at[pl.ds(8, CHUNK)])

    @pl.when(wid % (SEQ // CHUNK) != 0)
    def _():
        pltpu.sync_copy(ids_hbm.at[pl.ds(base - 8, CHUNK + 8)], ids_v)

    for g in range(GROUPS):
        cur = ids_v[pl.ds(8 + g * 16, 16)]
        prev = ids_v[pl.ds(7 + g * 16, 16)]
        x = (prev * P1C) ^ cur
        r = lax.rem(x, HASH_N)
        h = jnp.where(r < 0, r + HASH_N, r)
        h_v[g // (ROWS_PER_DMA // 16), pl.ds((g % (ROWS_PER_DMA // 16)) * 16, 16)] = h

    cps = [
        pltpu.async_copy(
            table_hbm.at[h_v.at[j]],
            rows_v.at[pl.ds(j * ROWS_PER_DMA, ROWS_PER_DMA)],
            sem,
        )
        for j in range(NDMA)
    ]
    for cp in cps:
        cp.wait()
    pltpu.sync_copy(rows_v, emb_hbm.at[pl.ds(base, CHUNK)])


def _sc_gather(ids_flat, table):
    mesh = plsc.VectorSubcoreMesh(core_axis_name="c", subcore_axis_name="s")
    fn = functools.partial(
        pl.kernel,
        mesh=mesh,
        out_type=jax.ShapeDtypeStruct((NTOK, EMB), jnp.float32),
        scratch_types=[
            pltpu.VMEM((CHUNK + 8,), jnp.int32),
            pltpu.VMEM((NDMA, ROWS_PER_DMA), jnp.int32),
            pltpu.VMEM((CHUNK, EMB), jnp.float32),
            pltpu.SemaphoreType.DMA,
        ],
    )(_sc_gather_kernel)
    return fn(ids_flat, table)


def _mm_body(x_ref, w_ref, o_ref):
    o_ref[...] = lax.dot_general(
        x_ref[...].astype(jnp.bfloat16),
        w_ref[...].astype(jnp.bfloat16),
        dimension_numbers=(((1,), (1,)), ((), ())),
        preferred_element_type=jnp.float32,
    )


def _project(emb, proj_w):
    return pl.pallas_call(
        _mm_body,
        grid=(NTOK // BT,),
        in_specs=[
            pl.BlockSpec((BT, EMB), lambda i: (i, 0)),
            pl.BlockSpec((DM, EMB), lambda i: (0, 0)),
        ],
        out_specs=pl.BlockSpec((BT, DM), lambda i: (i, 0)),
        out_shape=jax.ShapeDtypeStruct((NTOK, DM), jnp.float32),
    )(emb, proj_w)


@jax.jit
def kernel(input_ids, bigram_emb, proj_w):
    ids_flat = input_ids.reshape(-1)
    emb = _sc_gather(ids_flat, bigram_emb)
    out = _project(emb, proj_w)
    return out.reshape(BATCH, SEQ, DM)
